# baseline (speedup 1.0000x reference)
import functools

import jax
import jax.numpy as jnp
from jax import lax
from jax.experimental import pallas as pl
from jax.experimental.pallas import tpu as pltpu
from jax.experimental.pallas import tpu_sc as plsc

N = 32768
C = 2048
NC, NS, L = 2, 16, 16
NW = NC * NS
PER_W = N // NW
CHUNK = 128
NCHUNK = PER_W // CHUNK
VPC = CHUNK // L


def _sc_partials(flat_logits, target):
    mesh = plsc.VectorSubcoreMesh(core_axis_name="c", subcore_axis_name="s")

    @functools.partial(
        pl.kernel,
        out_type=jax.ShapeDtypeStruct((NW * 2 * L,), jnp.float32),
        mesh=mesh,
        scratch_types=[
            pltpu.VMEM((PER_W,), jnp.int32),
            pltpu.VMEM((NCHUNK, CHUNK), jnp.int32),
            pltpu.VMEM((NCHUNK, CHUNK), jnp.float32),
            pltpu.VMEM((2 * L,), jnp.float32),
            pltpu.SemaphoreType.DMA,
        ],
    )
    def kern(logits_hbm, tgt_hbm, out_hbm, tgt_v, idx_v, gat_v, acc_v, sem):
        wid = lax.axis_index("s") * NC + lax.axis_index("c")
        base = wid * PER_W

        pltpu.sync_copy(tgt_hbm.at[pl.ds(base, PER_W)], tgt_v)

        lane = lax.iota(jnp.int32, L)
        for ch in range(NCHUNK):
            for v in range(VPC):
                off = ch * CHUNK + v * L
                t16 = tgt_v[pl.ds(off, L)]
                safe = jnp.where(t16 >= 0, t16, 0)
                row = base + off + lane
                idx_v[ch, pl.ds(v * L, L)] = row * C + safe

        copies = [
            pltpu.make_async_copy(logits_hbm.at[idx_v.at[ch]], gat_v.at[ch], sem)
            for ch in range(NCHUNK)
        ]
        for cp in copies:
            cp.start()
        for cp in copies:
            cp.wait()

        acc_sq = jnp.zeros((L,), jnp.float32)
        acc_m = jnp.zeros((L,), jnp.float32)
        for ch in range(NCHUNK):
            for v in range(VPC):
                off = ch * CHUNK + v * L
                t16 = tgt_v[pl.ds(off, L)]
                m16 = jnp.where(t16 >= 0, 1.0, 0.0).astype(jnp.float32)
                g16 = gat_v[ch, pl.ds(v * L, L)]
                d = 1.0 - g16
                acc_sq = acc_sq + d * d * m16
                acc_m = acc_m + m16

        acc_v[pl.ds(0, L)] = acc_sq
        acc_v[pl.ds(L, L)] = acc_m
        pltpu.sync_copy(acc_v.at[pl.ds(0, L)], out_hbm.at[pl.ds(wid * L, L)])
        pltpu.sync_copy(
            acc_v.at[pl.ds(L, L)], out_hbm.at[pl.ds(NW * L + wid * L, L)]
        )

    return kern(flat_logits, target)


def _tc_finalize(partials):
    def body(p_ref, o_ref):
        p = p_ref[...]
        s = jnp.sum(p[0:4])
        m = jnp.sum(p[4:8])
        o_ref[...] = jnp.full((1, 1), s / m, jnp.float32)

    return pl.pallas_call(
        body,
        out_shape=jax.ShapeDtypeStruct((1, 1), jnp.float32),
    )(partials)


@jax.jit
def kernel(contrast_logits, contrast_target):
    flat = contrast_logits.reshape(-1)
    partials = _sc_partials(flat, contrast_target)
    loss = _tc_finalize(partials.reshape(8, 128))
    return loss[0, 0]
